# baseline (speedup 1.0000x reference)
import dataclasses

import jax
import jax.numpy as jnp
from jax import lax
from jax.experimental import pallas as pl
from jax.experimental.pallas import tpu as pltpu
from jax.experimental.pallas import tpu_sc as plsc

_RROWS = 1024
_TPAD = 1024
_P1 = 9216
_KSC = 3072
_BR = 8


def _qsample_body(t_ref, a_ref, b_ref, x_ref, n_ref, o_ref, coef_ref):
    @pl.when(pl.program_id(0) == 0)
    def _():
        t_row = t_ref[...]
        ids = jax.lax.broadcasted_iota(jnp.int32, (_TPAD, t_row.shape[1]), 0)
        m = ids == t_row
        zero = jnp.zeros((), jnp.float32)
        coef_ref[0:1, :] = jnp.sum(jnp.where(m, a_ref[...], zero), axis=0, keepdims=True)
        coef_ref[1:2, :] = jnp.sum(jnp.where(m, b_ref[...], zero), axis=0, keepdims=True)

    ca = coef_ref[0:1, :]
    cb = coef_ref[1:2, :]
    o_ref[...] = ca * x_ref[...] + cb * n_ref[...]


def _sc_body(t_hbm, a_hbm, b_hbm, x_hbm, n_hbm, o_hbm, t_v, a_v, b_v, ca_v, cb_v):
    pltpu.sync_copy(t_hbm, t_v)
    pltpu.sync_copy(a_hbm, a_v)
    pltpu.sync_copy(b_hbm, b_v)

    @pl.loop(0, _TPAD, step=16)
    def _(c):
        sl = pl.ds(c, 16)
        idx = t_v[sl]
        ca_v[sl] = plsc.load_gather(a_v, [idx])
        cb_v[sl] = plsc.load_gather(b_v, [idx])

    def fma_block(x_vmem, n_vmem, o_vmem):
        @pl.loop(0, _TPAD, step=16)
        def _(c):
            sl = pl.ds(c, 16)
            ca = ca_v[sl]
            cb = cb_v[sl]

            @pl.loop(0, _BR)
            def _(r):
                o_vmem[r, sl] = ca * x_vmem[r, sl] + cb * n_vmem[r, sl]

    pltpu.emit_pipeline(
        fma_block,
        grid=(_KSC // _BR,),
        in_specs=[
            pl.BlockSpec((_BR, _TPAD), index_map=lambda i: (_P1 // _BR + i, 0)),
            pl.BlockSpec((_BR, _TPAD), index_map=lambda i: (_P1 // _BR + i, 0)),
        ],
        out_specs=[pl.BlockSpec((_BR, _TPAD), index_map=lambda i: (i, 0))],
        core_axis_name=("c", "s"),
        dimension_semantics=(pltpu.PARALLEL,),
    )(x_hbm, n_hbm, o_hbm)


def kernel(x_start, t, noise, sqrt_alphas_cumprod, sqrt_one_minus_alphas_cumprod):
    B, C, H, W = x_start.shape
    P = C * H * W
    xt = x_start.transpose(1, 2, 3, 0).reshape(P, B)
    nt = noise.transpose(1, 2, 3, 0).reshape(P, B)
    t1 = t.reshape(1, B)
    T = sqrt_alphas_cumprod.shape[0]
    a_pad = jnp.zeros((_TPAD,), jnp.float32).at[:T].set(sqrt_alphas_cumprod)
    b_pad = jnp.zeros((_TPAD,), jnp.float32).at[:T].set(sqrt_one_minus_alphas_cumprod)
    a_col = a_pad.reshape(_TPAD, 1)
    b_col = b_pad.reshape(_TPAD, 1)

    sc_params = pltpu.CompilerParams()
    if "needs_layout_passes" in pltpu.CompilerParams.__dataclass_fields__:
        sc_params = dataclasses.replace(sc_params, needs_layout_passes=False)
    mesh = plsc.VectorSubcoreMesh(core_axis_name="c", subcore_axis_name="s")
    sc_fma = pl.kernel(
        _sc_body,
        out_type=jax.ShapeDtypeStruct((_KSC, B), jnp.float32),
        mesh=mesh,
        scratch_types=[
            pltpu.VMEM((_TPAD,), jnp.int32),
            pltpu.VMEM((_TPAD,), jnp.float32),
            pltpu.VMEM((_TPAD,), jnp.float32),
            pltpu.VMEM((_TPAD,), jnp.float32),
            pltpu.VMEM((_TPAD,), jnp.float32),
        ],
        compiler_params=sc_params,
    )
    out_sc = sc_fma(t, a_pad, b_pad, xt, nt)

    out_tc = pl.pallas_call(
        _qsample_body,
        grid=(_P1 // _RROWS,),
        in_specs=[
            pl.BlockSpec((1, B), lambda i: (0, 0)),
            pl.BlockSpec((_TPAD, 1), lambda i: (0, 0)),
            pl.BlockSpec((_TPAD, 1), lambda i: (0, 0)),
            pl.BlockSpec((_RROWS, B), lambda i: (i, 0)),
            pl.BlockSpec((_RROWS, B), lambda i: (i, 0)),
        ],
        out_specs=pl.BlockSpec((_RROWS, B), lambda i: (i, 0)),
        out_shape=jax.ShapeDtypeStruct((P, B), jnp.float32),
        scratch_shapes=[pltpu.VMEM((2, B), jnp.float32)],
        compiler_params=pltpu.CompilerParams(dimension_semantics=("arbitrary",)),
    )(t1, a_col, b_col, xt, nt)

    out = lax.dynamic_update_slice(out_tc, out_sc, (_P1, 0))
    return out.reshape(C, H, W, B).transpose(3, 0, 1, 2)


# speedup vs baseline: 1.0937x; 1.0937x over previous
import dataclasses

import jax
import jax.numpy as jnp
from jax import lax
from jax.experimental import pallas as pl
from jax.experimental.pallas import tpu as pltpu
from jax.experimental.pallas import tpu_sc as plsc

_RROWS = 1024
_TPAD = 1024
_P1 = 10240
_KSC = 2048
_BR = 8


def _qsample_body(t_ref, a_ref, b_ref, x_ref, n_ref, o_ref, coef_ref):
    @pl.when(pl.program_id(0) == 0)
    def _():
        t_row = t_ref[...]
        ids = jax.lax.broadcasted_iota(jnp.int32, (_TPAD, t_row.shape[1]), 0)
        m = ids == t_row
        zero = jnp.zeros((), jnp.float32)
        coef_ref[0:1, :] = jnp.sum(jnp.where(m, a_ref[...], zero), axis=0, keepdims=True)
        coef_ref[1:2, :] = jnp.sum(jnp.where(m, b_ref[...], zero), axis=0, keepdims=True)

    ca = coef_ref[0:1, :]
    cb = coef_ref[1:2, :]
    o_ref[...] = ca * x_ref[...] + cb * n_ref[...]


def _sc_body(t_hbm, a_hbm, b_hbm, x_hbm, n_hbm, o_hbm, t_v, a_v, b_v, ca_v, cb_v):
    pltpu.sync_copy(t_hbm, t_v)
    pltpu.sync_copy(a_hbm, a_v)
    pltpu.sync_copy(b_hbm, b_v)

    @pl.loop(0, _TPAD, step=16)
    def _(c):
        sl = pl.ds(c, 16)
        idx = t_v[sl]
        ca_v[sl] = plsc.load_gather(a_v, [idx])
        cb_v[sl] = plsc.load_gather(b_v, [idx])

    def fma_block(x_vmem, n_vmem, o_vmem):
        @pl.loop(0, _TPAD, step=16)
        def _(c):
            sl = pl.ds(c, 16)
            ca = ca_v[sl]
            cb = cb_v[sl]
            for r in range(_BR):
                o_vmem[r, sl] = ca * x_vmem[r, sl] + cb * n_vmem[r, sl]

    pltpu.emit_pipeline(
        fma_block,
        grid=(_KSC // _BR,),
        in_specs=[
            pl.BlockSpec((_BR, _TPAD), index_map=lambda i: (_P1 // _BR + i, 0)),
            pl.BlockSpec((_BR, _TPAD), index_map=lambda i: (_P1 // _BR + i, 0)),
        ],
        out_specs=[pl.BlockSpec((_BR, _TPAD), index_map=lambda i: (i, 0))],
        core_axis_name=("c", "s"),
        dimension_semantics=(pltpu.PARALLEL,),
    )(x_hbm, n_hbm, o_hbm)


def kernel(x_start, t, noise, sqrt_alphas_cumprod, sqrt_one_minus_alphas_cumprod):
    B, C, H, W = x_start.shape
    P = C * H * W
    xt = x_start.transpose(1, 2, 3, 0).reshape(P, B)
    nt = noise.transpose(1, 2, 3, 0).reshape(P, B)
    t1 = t.reshape(1, B)
    T = sqrt_alphas_cumprod.shape[0]
    a_pad = jnp.zeros((_TPAD,), jnp.float32).at[:T].set(sqrt_alphas_cumprod)
    b_pad = jnp.zeros((_TPAD,), jnp.float32).at[:T].set(sqrt_one_minus_alphas_cumprod)
    a_col = a_pad.reshape(_TPAD, 1)
    b_col = b_pad.reshape(_TPAD, 1)

    sc_params = pltpu.CompilerParams()
    if "needs_layout_passes" in pltpu.CompilerParams.__dataclass_fields__:
        sc_params = dataclasses.replace(sc_params, needs_layout_passes=False)
    mesh = plsc.VectorSubcoreMesh(core_axis_name="c", subcore_axis_name="s")
    sc_fma = pl.kernel(
        _sc_body,
        out_type=jax.ShapeDtypeStruct((_KSC, B), jnp.float32),
        mesh=mesh,
        scratch_types=[
            pltpu.VMEM((_TPAD,), jnp.int32),
            pltpu.VMEM((_TPAD,), jnp.float32),
            pltpu.VMEM((_TPAD,), jnp.float32),
            pltpu.VMEM((_TPAD,), jnp.float32),
            pltpu.VMEM((_TPAD,), jnp.float32),
        ],
        compiler_params=sc_params,
    )
    out_sc = sc_fma(t, a_pad, b_pad, xt, nt)

    out_tc = pl.pallas_call(
        _qsample_body,
        grid=(_P1 // _RROWS,),
        in_specs=[
            pl.BlockSpec((1, B), lambda i: (0, 0)),
            pl.BlockSpec((_TPAD, 1), lambda i: (0, 0)),
            pl.BlockSpec((_TPAD, 1), lambda i: (0, 0)),
            pl.BlockSpec((_RROWS, B), lambda i: (i, 0)),
            pl.BlockSpec((_RROWS, B), lambda i: (i, 0)),
        ],
        out_specs=pl.BlockSpec((_RROWS, B), lambda i: (i, 0)),
        out_shape=jax.ShapeDtypeStruct((P, B), jnp.float32),
        scratch_shapes=[pltpu.VMEM((2, B), jnp.float32)],
        compiler_params=pltpu.CompilerParams(dimension_semantics=("arbitrary",)),
    )(t1, a_col, b_col, xt, nt)

    out = lax.dynamic_update_slice(out_tc, out_sc, (_P1, 0))
    return out.reshape(C, H, W, B).transpose(3, 0, 1, 2)


# speedup vs baseline: 1.7346x; 1.5860x over previous
import jax
import jax.numpy as jnp
from jax.experimental import pallas as pl
from jax.experimental.pallas import tpu as pltpu

_RROWS = 2048
_TPAD = 1024


def _qsample_body(t_ref, ab_ref, x_ref, n_ref, o_ref, coef_ref):
    @pl.when(pl.program_id(0) == 0)
    def _():
        t_row = t_ref[...]
        ids = jax.lax.broadcasted_iota(jnp.int32, (_TPAD, t_row.shape[1]), 0)
        onehot = jnp.where(ids == t_row, 1.0, 0.0)
        coef_ref[...] = jax.lax.dot_general(
            ab_ref[...], onehot,
            (((1,), (0,)), ((), ())),
            preferred_element_type=jnp.float32,
        )

    ca = coef_ref[0:1, :]
    cb = coef_ref[1:2, :]
    o_ref[...] = ca * x_ref[...] + cb * n_ref[...]


def kernel(x_start, t, noise, sqrt_alphas_cumprod, sqrt_one_minus_alphas_cumprod):
    B, C, H, W = x_start.shape
    P = C * H * W
    xt = x_start.transpose(1, 2, 3, 0).reshape(P, B)
    nt = noise.transpose(1, 2, 3, 0).reshape(P, B)
    t1 = t.reshape(1, B)
    T = sqrt_alphas_cumprod.shape[0]
    ab = (
        jnp.zeros((2, _TPAD), jnp.float32)
        .at[0, :T].set(sqrt_alphas_cumprod)
        .at[1, :T].set(sqrt_one_minus_alphas_cumprod)
    )

    out = pl.pallas_call(
        _qsample_body,
        grid=(P // _RROWS,),
        in_specs=[
            pl.BlockSpec((1, B), lambda i: (0, 0)),
            pl.BlockSpec((2, _TPAD), lambda i: (0, 0)),
            pl.BlockSpec((_RROWS, B), lambda i: (i, 0)),
            pl.BlockSpec((_RROWS, B), lambda i: (i, 0)),
        ],
        out_specs=pl.BlockSpec((_RROWS, B), lambda i: (i, 0)),
        out_shape=jax.ShapeDtypeStruct((P, B), jnp.float32),
        scratch_shapes=[pltpu.VMEM((2, B), jnp.float32)],
        compiler_params=pltpu.CompilerParams(dimension_semantics=("arbitrary",)),
    )(t1, ab, xt, nt)
    return out.reshape(C, H, W, B).transpose(3, 0, 1, 2)
